# baseline (speedup 1.0000x reference)
import functools
import jax
import jax.numpy as jnp
from jax import lax
from jax.experimental import pallas as pl
from jax.experimental.pallas import tpu as pltpu, tpu_sc as plsc


def kernel(x, pe):
    B, S, D = x.shape
    NC, NS = 2, 16
    NW = NC * NS
    N = B * S * D
    PER_W = N // NW
    CH = 32768
    N_CHUNKS = PER_W // CH
    S_ELEMS = S * D

    x_flat = x.reshape(N)
    pe_flat = pe[:S].reshape(S_ELEMS)

    mesh = plsc.VectorSubcoreMesh(core_axis_name="c", subcore_axis_name="s")

    @functools.partial(
        pl.kernel,
        mesh=mesh,
        out_type=jax.ShapeDtypeStruct((N,), jnp.float32),
        scratch_types=[
            pltpu.VMEM((CH,), jnp.float32),
            pltpu.VMEM((CH,), jnp.float32),
        ],
    )
    def k(x_hbm, pe_hbm, o_hbm, xbuf, pebuf):
        wid = lax.axis_index("s") * NC + lax.axis_index("c")
        base = wid * PER_W
        pe_base = lax.rem(base, S_ELEMS)

        def chunk_body(ci, _):
            off = base + ci * CH
            pe_off = pe_base + ci * CH
            pltpu.sync_copy(x_hbm.at[pl.ds(off, CH)], xbuf)
            pltpu.sync_copy(pe_hbm.at[pl.ds(pe_off, CH)], pebuf)

            def vec_body(i, _):
                sl = pl.ds(i * 16, 16)
                xbuf[sl] = xbuf[sl] + pebuf[sl]
                return 0

            lax.fori_loop(0, CH // 16, vec_body, 0)
            pltpu.sync_copy(xbuf, o_hbm.at[pl.ds(off, CH)])
            return 0

        lax.fori_loop(0, N_CHUNKS, chunk_body, 0)

    out = k(x_flat, pe_flat)
    return out.reshape(B, S, D)


# speedup vs baseline: 1.4271x; 1.4271x over previous
import functools
import jax
import jax.numpy as jnp
from jax import lax
from jax.experimental import pallas as pl
from jax.experimental.pallas import tpu as pltpu, tpu_sc as plsc


def kernel(x, pe):
    B, S, D = x.shape
    NC, NS = 2, 16
    NW = NC * NS
    N = B * S * D
    PER_W = N // NW
    CH = 32768
    N_CHUNKS = PER_W // CH
    S_ELEMS = S * D

    x_flat = x.reshape(N)
    pe_flat = pe[:S].reshape(S_ELEMS)

    mesh = plsc.VectorSubcoreMesh(core_axis_name="c", subcore_axis_name="s")

    @functools.partial(
        pl.kernel,
        mesh=mesh,
        out_type=jax.ShapeDtypeStruct((N,), jnp.float32),
        scratch_types=[
            pltpu.VMEM((CH,), jnp.float32),
            pltpu.VMEM((CH,), jnp.float32),
        ],
    )
    def k(x_hbm, pe_hbm, o_hbm, xbuf, pebuf):
        wid = lax.axis_index("s") * NC + lax.axis_index("c")
        base = wid * PER_W
        pe_base = lax.rem(base, S_ELEMS)

        def chunk_body(ci, _):
            off = base + ci * CH
            pe_off = pe_base + ci * CH
            pltpu.sync_copy(x_hbm.at[pl.ds(off, CH)], xbuf)
            pltpu.sync_copy(pe_hbm.at[pl.ds(pe_off, CH)], pebuf)

            @plsc.parallel_loop(0, CH // 16, unroll=8)
            def vec_body(i):
                sl = pl.ds(i * 16, 16)
                xbuf[sl] = xbuf[sl] + pebuf[sl]
            pltpu.sync_copy(xbuf, o_hbm.at[pl.ds(off, CH)])
            return 0

        lax.fori_loop(0, N_CHUNKS, chunk_body, 0)

    out = k(x_flat, pe_flat)
    return out.reshape(B, S, D)
